# baseline (speedup 1.0000x reference)
import functools

import jax
import jax.numpy as jnp
from jax import lax
from jax.experimental import pallas as pl
from jax.experimental.pallas import tpu as pltpu
from jax.experimental.pallas import tpu_sc as plsc


@functools.cache
def _sc_gather(n_rows: int, hidden: int):
    info = plsc.get_sparse_core_info()
    nw = info.num_cores * info.num_subcores
    rows_per_w = n_rows // nw
    chunk = rows_per_w
    while chunk * hidden * 4 > 128 * 1024:
        chunk //= 2
    n_chunks = rows_per_w // chunk
    mesh = plsc.VectorSubcoreMesh(core_axis_name="c", subcore_axis_name="s")

    @functools.partial(
        pl.kernel,
        mesh=mesh,
        out_type=jax.ShapeDtypeStruct((n_rows, hidden), jnp.float32),
        scratch_types=[
            pltpu.VMEM((rows_per_w,), jnp.int32),
            pltpu.VMEM((chunk, hidden), jnp.float32),
            pltpu.VMEM((chunk, hidden), jnp.float32),
            pltpu.SemaphoreType.DMA,
            pltpu.SemaphoreType.DMA,
        ],
    )
    def gather(table_hbm, idx_hbm, out_hbm, idx_v, rows_a, rows_b, sem_a, sem_b):
        wid = lax.axis_index("s") * info.num_cores + lax.axis_index("c")
        base = wid * rows_per_w
        pltpu.sync_copy(idx_hbm.at[pl.ds(base, rows_per_w)], idx_v)
        bufs = ((rows_a, sem_a), (rows_b, sem_b))
        cps = [None, None]
        for c in range(n_chunks):
            buf, sem = bufs[c % 2]
            cps[c % 2] = pltpu.async_copy(
                table_hbm.at[idx_v.at[pl.ds(c * chunk, chunk)]], buf, sem)
            if c >= 1:
                pbuf, _ = bufs[(c - 1) % 2]
                cps[(c - 1) % 2].wait()
                pltpu.sync_copy(pbuf, out_hbm.at[pl.ds(base + (c - 1) * chunk, chunk)])
        lbuf, _ = bufs[(n_chunks - 1) % 2]
        cps[(n_chunks - 1) % 2].wait()
        pltpu.sync_copy(lbuf, out_hbm.at[pl.ds(base + (n_chunks - 1) * chunk, chunk)])

    return gather


def _fused_body(x_ref, w_ref, b_ref, lbl_ref, logits_ref, loss_ref,
                m_ref, s_ref, ll_ref, acc_ref, cnt_ref, *, tm, tn, nj, ni):
    j = pl.program_id(0)
    i = pl.program_id(1)
    rows = pl.ds(i * tm, tm)
    nk = tn // 128

    acc = jnp.dot(x_ref[rows, :].astype(jnp.bfloat16),
                  w_ref[...].astype(jnp.bfloat16),
                  preferred_element_type=jnp.float32) + b_ref[...]
    logits_ref[...] = acc

    acc3 = acc.reshape(tm, nk, 128)
    tmax = jnp.max(acc3, axis=1)
    lbl = lbl_ref[...]
    local = lbl - j * tn
    col = lax.broadcasted_iota(jnp.int32, (tm, tn), 1)
    hit = jnp.where(col == local, acc, 0.0)
    pick = jnp.sum(hit.reshape(tm, nk, 128), axis=1)

    m_old = jnp.where(j == 0, -3e38, m_ref[rows, :])
    s_old = jnp.where(j == 0, 0.0, s_ref[rows, :])
    ll_old = jnp.where(j == 0, 0.0, ll_ref[rows, :])
    m_new = jnp.maximum(m_old, tmax)
    e_sum = jnp.sum(jnp.exp(acc3 - m_new[:, None, :]), axis=1)
    s_new = s_old * jnp.exp(m_old - m_new) + e_sum
    ll_new = ll_old + pick
    m_ref[rows, :] = m_new
    s_ref[rows, :] = s_new
    ll_ref[rows, :] = ll_new

    @pl.when(j == nj - 1)
    def _():
        @pl.when(i == 0)
        def _():
            acc_ref[...] = jnp.zeros_like(acc_ref)
            cnt_ref[...] = jnp.zeros_like(cnt_ref)

        mrow = jnp.max(m_new, axis=1, keepdims=True)
        srow = jnp.sum(s_new * jnp.exp(m_new - mrow), axis=1,
                       keepdims=True)
        llrow = jnp.sum(ll_new, axis=1, keepdims=True)
        valid = lbl >= 0
        nll = mrow + jnp.log(srow) - llrow
        acc_ref[...] += jnp.sum(jnp.where(valid, nll, 0.0), axis=(0, 1),
                                keepdims=True)
        cnt_ref[...] += jnp.sum(jnp.where(valid, 1.0, 0.0), axis=(0, 1),
                                keepdims=True)

        @pl.when(i == ni - 1)
        def _():
            loss_ref[...] = acc_ref[...] / jnp.maximum(cnt_ref[...], 1.0)


@functools.cache
def _fused_call(nt: int, hidden: int, vocab: int, tm: int, tn: int,
                interpret: bool = False):
    nj = vocab // tn
    ni = nt // tm
    return pl.pallas_call(
        functools.partial(_fused_body, tm=tm, tn=tn, nj=nj, ni=ni),
        grid=(nj, ni),
        in_specs=[
            pl.BlockSpec((nt, hidden), lambda j, i: (0, 0)),
            pl.BlockSpec((hidden, tn), lambda j, i: (0, j)),
            pl.BlockSpec((1, tn), lambda j, i: (0, j)),
            pl.BlockSpec((tm, 1), lambda j, i: (i, 0)),
        ],
        out_specs=[
            pl.BlockSpec((tm, tn), lambda j, i: (i, j)),
            pl.BlockSpec((1, 1), lambda j, i: (0, 0)),
        ],
        out_shape=[
            jax.ShapeDtypeStruct((nt, vocab), jnp.float32),
            jax.ShapeDtypeStruct((1, 1), jnp.float32),
        ],
        scratch_shapes=[
            pltpu.VMEM((nt, 128), jnp.float32),
            pltpu.VMEM((nt, 128), jnp.float32),
            pltpu.VMEM((nt, 128), jnp.float32),
            pltpu.VMEM((1, 1), jnp.float32),
            pltpu.VMEM((1, 1), jnp.float32),
        ],
        compiler_params=pltpu.CompilerParams(
            dimension_semantics=("arbitrary", "arbitrary"),
        ),
        interpret=interpret,
    )


def kernel(input_ids, labels, embed, W, b):
    bsz, t = input_ids.shape
    vocab, hidden = embed.shape
    nt = bsz * t

    ids = input_ids.reshape(-1).astype(jnp.int32)
    x = _sc_gather(nt, hidden)(embed, ids)

    shifted = jnp.concatenate(
        [labels[:, 1:], jnp.full((bsz, 1), -1, labels.dtype)], axis=1)
    shifted = shifted.reshape(nt, 1).astype(jnp.int32)

    logits_flat, loss = _fused_call(nt, hidden, vocab, 512, 1280)(
        x, W, b.reshape(1, vocab), shifted)
    return (loss.reshape(()), logits_flat.reshape(bsz, t, vocab))


# speedup vs baseline: 2.3218x; 2.3218x over previous
import functools

import jax
import jax.numpy as jnp
from jax import lax
from jax.experimental import pallas as pl
from jax.experimental.pallas import tpu as pltpu
from jax.experimental.pallas import tpu_sc as plsc


@functools.cache
def _sc_gather(n_rows: int, hidden: int):
    info = plsc.get_sparse_core_info()
    nw = info.num_cores * info.num_subcores
    rows_per_w = n_rows // nw
    chunk = rows_per_w
    while chunk * hidden * 4 > 128 * 1024:
        chunk //= 2
    n_chunks = rows_per_w // chunk
    mesh = plsc.VectorSubcoreMesh(core_axis_name="c", subcore_axis_name="s")

    @functools.partial(
        pl.kernel,
        mesh=mesh,
        out_type=jax.ShapeDtypeStruct((n_rows, hidden), jnp.float32),
        scratch_types=[
            pltpu.VMEM((rows_per_w,), jnp.int32),
            pltpu.VMEM((chunk, hidden), jnp.float32),
            pltpu.VMEM((chunk, hidden), jnp.float32),
            pltpu.SemaphoreType.DMA,
            pltpu.SemaphoreType.DMA,
        ],
    )
    def gather(table_hbm, idx_hbm, out_hbm, idx_v, rows_a, rows_b, sem_a, sem_b):
        wid = lax.axis_index("s") * info.num_cores + lax.axis_index("c")
        base = wid * rows_per_w
        pltpu.sync_copy(idx_hbm.at[pl.ds(base, rows_per_w)], idx_v)
        bufs = ((rows_a, sem_a), (rows_b, sem_b))
        cps = [None, None]
        for c in range(n_chunks):
            buf, sem = bufs[c % 2]
            cps[c % 2] = pltpu.async_copy(
                table_hbm.at[idx_v.at[pl.ds(c * chunk, chunk)]], buf, sem)
            if c >= 1:
                pbuf, _ = bufs[(c - 1) % 2]
                cps[(c - 1) % 2].wait()
                pltpu.sync_copy(pbuf, out_hbm.at[pl.ds(base + (c - 1) * chunk, chunk)])
        lbuf, _ = bufs[(n_chunks - 1) % 2]
        cps[(n_chunks - 1) % 2].wait()
        pltpu.sync_copy(lbuf, out_hbm.at[pl.ds(base + (n_chunks - 1) * chunk, chunk)])

    return gather


def _fused_body(x_ref, w_ref, b_ref, lbl_ref, logits_ref, loss_ref,
                m_ref, s_ref, ll_ref, acc_ref, cnt_ref, *, tm, tn, nj, ni):
    j = pl.program_id(0)
    i = pl.program_id(1)
    rows = pl.ds(i * tm, tm)
    nk = tn // 128

    acc = jnp.dot(x_ref[rows, :].astype(jnp.bfloat16),
                  w_ref[...].astype(jnp.bfloat16),
                  preferred_element_type=jnp.float32) + b_ref[...]
    logits_ref[...] = acc

    tmax = jnp.max(acc, axis=1, keepdims=True)
    lbl = lbl_ref[...]
    local = lbl - j * tn
    col = lax.broadcasted_iota(jnp.int32, (tm, tn), 1)
    pick = jnp.sum(jnp.where(col == local, acc, 0.0), axis=1, keepdims=True)

    @pl.when(j == 0)
    def _():
        m_ref[rows, :] = tmax
        s_ref[rows, :] = jnp.sum(jnp.exp(acc - tmax), axis=1, keepdims=True)
        ll_ref[rows, :] = pick

    @pl.when(j > 0)
    def _():
        m_old = m_ref[rows, :]
        m_new = jnp.maximum(m_old, tmax)
        s_ref[rows, :] = (s_ref[rows, :] * jnp.exp(m_old - m_new)
                          + jnp.sum(jnp.exp(acc - m_new), axis=1, keepdims=True))
        m_ref[rows, :] = m_new
        ll_ref[rows, :] = ll_ref[rows, :] + pick

    @pl.when(j == nj - 1)
    def _():
        @pl.when(i == 0)
        def _():
            acc_ref[...] = jnp.zeros_like(acc_ref)
            cnt_ref[...] = jnp.zeros_like(cnt_ref)

        valid = lbl >= 0
        nll = m_ref[rows, :] + jnp.log(s_ref[rows, :]) - ll_ref[rows, :]
        acc_ref[...] += jnp.sum(jnp.where(valid, nll, 0.0), axis=(0, 1),
                                keepdims=True)
        cnt_ref[...] += jnp.sum(jnp.where(valid, 1.0, 0.0), axis=(0, 1),
                                keepdims=True)

        @pl.when(i == ni - 1)
        def _():
            loss_ref[...] = acc_ref[...] / jnp.maximum(cnt_ref[...], 1.0)


@functools.cache
def _fused_call(nt: int, hidden: int, vocab: int, tm: int, tn: int,
                interpret: bool = False):
    nj = vocab // tn
    ni = nt // tm
    return pl.pallas_call(
        functools.partial(_fused_body, tm=tm, tn=tn, nj=nj, ni=ni),
        grid=(nj, ni),
        in_specs=[
            pl.BlockSpec((nt, hidden), lambda j, i: (0, 0)),
            pl.BlockSpec((hidden, tn), lambda j, i: (0, j)),
            pl.BlockSpec((1, tn), lambda j, i: (0, j)),
            pl.BlockSpec((tm, 1), lambda j, i: (i, 0)),
        ],
        out_specs=[
            pl.BlockSpec((tm, tn), lambda j, i: (i, j)),
            pl.BlockSpec((1, 1), lambda j, i: (0, 0)),
        ],
        out_shape=[
            jax.ShapeDtypeStruct((nt, vocab), jnp.float32),
            jax.ShapeDtypeStruct((1, 1), jnp.float32),
        ],
        scratch_shapes=[
            pltpu.VMEM((nt, 1), jnp.float32),
            pltpu.VMEM((nt, 1), jnp.float32),
            pltpu.VMEM((nt, 1), jnp.float32),
            pltpu.VMEM((1, 1), jnp.float32),
            pltpu.VMEM((1, 1), jnp.float32),
        ],
        compiler_params=pltpu.CompilerParams(
            dimension_semantics=("arbitrary", "arbitrary"),
        ),
        interpret=interpret,
    )


def kernel(input_ids, labels, embed, W, b):
    bsz, t = input_ids.shape
    vocab, hidden = embed.shape
    nt = bsz * t

    ids = input_ids.reshape(-1).astype(jnp.int32)
    x = _sc_gather(nt, hidden)(embed, ids)

    shifted = jnp.concatenate(
        [labels[:, 1:], jnp.full((bsz, 1), -1, labels.dtype)], axis=1)
    shifted = shifted.reshape(nt, 1).astype(jnp.int32)

    logits_flat, loss = _fused_call(nt, hidden, vocab, 1024, 1280)(
        x, W, b.reshape(1, vocab), shifted)
    return (loss.reshape(()), logits_flat.reshape(bsz, t, vocab))
